# Initial kernel scaffold; baseline (speedup 1.0000x reference)
import jax
import jax.numpy as jnp
from jax.experimental import pallas as pl


def kernel(x, emb):
    raise NotImplementedError("write your pallas kernel here")


# speedup vs baseline: 1.9936x; 1.9936x over previous
import jax
import jax.numpy as jnp
from jax import lax
from jax.experimental import pallas as pl
from jax.experimental.pallas import tpu as pltpu, tpu_sc as plsc

N_NODES = 100000
N_FEATS = 9
VOCAB = 100
HIDDEN = 256

NC = 2
NS = 16
NW = NC * NS

C = 32
FLAT = C * N_FEATS
NCHUNKS = N_NODES // C
T = (NCHUNKS + NW - 1) // NW

_mesh = plsc.VectorSubcoreMesh(core_axis_name="c", subcore_axis_name="s")


def _body(x_hbm, emb_hbm, out_hbm, x_v, offs_v, idx0, idx1, idx2,
          rows_v, out_v, sem):
    c = lax.axis_index("c")
    s = lax.axis_index("s")
    wid = s * NC + c
    idxs = (idx0, idx1, idx2)

    for k in range(FLAT // 16):
        j = lax.iota(jnp.int32, 16) + (16 * k)
        q = lax.shift_right_logical(j * 7282, 16)
        offs_v[pl.ds(16 * k, 16)] = (j - q * N_FEATS) * VOCAB

    def step(t, carry):
        g = wid + NW * t

        @pl.when(g < NCHUNKS)
        def _():
            pltpu.sync_copy(x_hbm.at[pl.ds(g * FLAT, FLAT)], x_v)
            for b in range(3):
                for k in range(6):
                    p = b * 96 + k * 16
                    idxs[b][pl.ds(16 * k, 16)] = (
                        x_v[pl.ds(p, 16)] + offs_v[pl.ds(p, 16)])
            cps = [pltpu.async_copy(emb_hbm.at[idxs[b]],
                                    rows_v.at[pl.ds(b * 96, 96)], sem)
                   for b in range(3)]
            for cp in cps:
                cp.wait()

            def node_sum(n, carry2):
                r = n * N_FEATS
                for k in range(HIDDEN // 16):
                    d = pl.ds(16 * k, 16)
                    acc = rows_v[r, d]
                    for f in range(1, N_FEATS):
                        acc = acc + rows_v[r + f, d]
                    out_v[n, d] = acc
                return carry2

            lax.fori_loop(0, C, node_sum, 0)
            pltpu.sync_copy(out_v, out_hbm.at[pl.ds(g * C, C)])
        return carry

    lax.fori_loop(0, T, step, 0)


_sc_call = pl.kernel(
    _body,
    out_type=jax.ShapeDtypeStruct((N_NODES, HIDDEN), jnp.float32),
    mesh=_mesh,
    scratch_types=[
        pltpu.VMEM((FLAT,), jnp.int32),
        pltpu.VMEM((FLAT,), jnp.int32),
        pltpu.VMEM((96,), jnp.int32),
        pltpu.VMEM((96,), jnp.int32),
        pltpu.VMEM((96,), jnp.int32),
        pltpu.VMEM((FLAT, HIDDEN), jnp.float32),
        pltpu.VMEM((C, HIDDEN), jnp.float32),
        pltpu.SemaphoreType.DMA,
    ],
)


def kernel(x, emb):
    x_flat = x.reshape(-1)
    emb_flat = emb.reshape(N_FEATS * VOCAB, HIDDEN)
    return _sc_call(x_flat, emb_flat)


# speedup vs baseline: 4.2297x; 2.1216x over previous
import jax
import jax.numpy as jnp
from jax import lax
from jax.experimental import pallas as pl
from jax.experimental.pallas import tpu as pltpu, tpu_sc as plsc

N_NODES = 100000
N_FEATS = 9
VOCAB = 100
HIDDEN = 256
NPAIR = 5
TBL_ROWS = 4 * VOCAB * VOCAB + VOCAB

NC = 2
NS = 16
NW = NC * NS

C = 32
SB = 8
NCHUNKS = N_NODES // C
T = (NCHUNKS + NW - 1) // NW
NSB = (T + SB - 1) // SB
XPAD = 512

_mesh = plsc.VectorSubcoreMesh(core_axis_name="c", subcore_axis_name="s")


def _body(xt_hbm, tbl_hbm, out_hbm, x_sb, idx0, idx1, idx2, idx3, idx4,
          rows_v, out_v, sem_x, sem_g):
    c = lax.axis_index("c")
    s = lax.axis_index("s")
    wid = s * NC + c
    g_start = wid * T
    idxs = (idx0, idx1, idx2, idx3, idx4)

    def superblock(u, carry):
        g0 = g_start + u * SB

        @pl.when(g0 < NCHUNKS)
        def _():
            nbase = g0 * C
            xcps = [pltpu.async_copy(
                xt_hbm.at[pl.ds(f * N_NODES + nbase, SB * C)],
                x_sb.at[pl.ds(f * (SB * C), SB * C)], sem_x)
                for f in range(N_FEATS)]
            for cp in xcps:
                cp.wait()

            for v in range(SB):
                g = g0 + v

                @pl.when(g < NCHUNKS)
                def _():
                    for p in range(4):
                        for k in range(2):
                            da = pl.ds((2 * p) * (SB * C) + 32 * v + 16 * k, 16)
                            db = pl.ds((2 * p + 1) * (SB * C) + 32 * v + 16 * k, 16)
                            idxs[p][pl.ds(16 * k, 16)] = (
                                x_sb[da] * VOCAB + x_sb[db]
                                + p * (VOCAB * VOCAB))
                    for k in range(2):
                        d8 = pl.ds(8 * (SB * C) + 32 * v + 16 * k, 16)
                        idxs[4][pl.ds(16 * k, 16)] = (
                            x_sb[d8] + 4 * (VOCAB * VOCAB))
                    cps = [pltpu.async_copy(tbl_hbm.at[idxs[p]],
                                            rows_v.at[pl.ds(32 * p, 32)],
                                            sem_g)
                           for p in range(NPAIR)]
                    for cp in cps:
                        cp.wait()

                    def node_sum(n, carry2):
                        for k in range(HIDDEN // 16):
                            d = pl.ds(16 * k, 16)
                            acc = rows_v[n, d]
                            for p in range(1, NPAIR):
                                acc = acc + rows_v[32 * p + n, d]
                            out_v[n, d] = acc
                        return carry2

                    lax.fori_loop(0, C, node_sum, 0)
                    pltpu.sync_copy(out_v, out_hbm.at[pl.ds(g * C, C)])
        return carry

    lax.fori_loop(0, NSB, superblock, 0)


_sc_call = pl.kernel(
    _body,
    out_type=jax.ShapeDtypeStruct((N_NODES, HIDDEN), jnp.float32),
    mesh=_mesh,
    scratch_types=[
        pltpu.VMEM((N_FEATS * SB * C,), jnp.int32),
        pltpu.VMEM((32,), jnp.int32),
        pltpu.VMEM((32,), jnp.int32),
        pltpu.VMEM((32,), jnp.int32),
        pltpu.VMEM((32,), jnp.int32),
        pltpu.VMEM((32,), jnp.int32),
        pltpu.VMEM((NPAIR * C, HIDDEN), jnp.float32),
        pltpu.VMEM((C, HIDDEN), jnp.float32),
        pltpu.SemaphoreType.DMA,
        pltpu.SemaphoreType.DMA,
    ],
)


def kernel(x, emb):
    pairs = [
        (emb[2 * p][:, None, :] + emb[2 * p + 1][None, :, :]).reshape(
            VOCAB * VOCAB, HIDDEN)
        for p in range(4)
    ]
    tbl = jnp.concatenate(pairs + [emb[8]], axis=0)
    xt = jnp.pad(x.T.reshape(-1), (0, XPAD))
    return _sc_call(xt, tbl)


# speedup vs baseline: 4.8421x; 1.1448x over previous
import jax
import jax.numpy as jnp
from jax import lax
from jax.experimental import pallas as pl
from jax.experimental.pallas import tpu as pltpu, tpu_sc as plsc

N_NODES = 100000
N_FEATS = 9
VOCAB = 100
HIDDEN = 256
NPAIR = 5
PAIRB = VOCAB * VOCAB

NC = 2
NS = 16
NW = NC * NS

C = 32
FLAT = C * N_FEATS
NCHUNKS = N_NODES // C
T = (NCHUNKS + NW - 1) // NW
NITER = (T + 1) // 2

_mesh = plsc.VectorSubcoreMesh(core_axis_name="c", subcore_axis_name="s")


def _stage_x(xc_hbm, xbuf, g, sem):
    return pltpu.async_copy(xc_hbm.at[pl.ds(g * FLAT, FLAT)], xbuf, sem)


def _compute_idx(xbuf, idxs):
    for p in range(4):
        for k in range(2):
            da = pl.ds((2 * p) * C + 16 * k, 16)
            db = pl.ds((2 * p + 1) * C + 16 * k, 16)
            idxs[p][pl.ds(16 * k, 16)] = (
                xbuf[da] * VOCAB + xbuf[db] + p * PAIRB)
    for k in range(2):
        d8 = pl.ds(8 * C + 16 * k, 16)
        idxs[4][pl.ds(16 * k, 16)] = xbuf[d8] + 4 * PAIRB


def _issue_gathers(tbl_hbm, idxs, rows, sem):
    return [pltpu.async_copy(tbl_hbm.at[idxs[p]],
                             rows.at[pl.ds(C * p, C)], sem)
            for p in range(NPAIR)]


def _wait_gathers(tbl_hbm, idxs, rows, sem):
    for p in range(NPAIR):
        pltpu.make_async_copy(tbl_hbm.at[idxs[p]],
                              rows.at[pl.ds(C * p, C)], sem).wait()


def _sum_chunk(rows, out_v):
    def node_sum(n, carry):
        for k in range(HIDDEN // 16):
            d = pl.ds(16 * k, 16)
            acc = rows[n, d]
            for p in range(1, NPAIR):
                acc = acc + rows[C * p + n, d]
            out_v[n, d] = acc
        return carry

    lax.fori_loop(0, C, node_sum, 0)


def _body(xc_hbm, tbl_hbm, out_hbm, xA, xB, iA0, iA1, iA2, iA3, iA4,
          iB0, iB1, iB2, iB3, iB4, rowsA, rowsB, outA, outB,
          sem_x, sem_g, sem_oA, sem_oB):
    c = lax.axis_index("c")
    s = lax.axis_index("s")
    wid = s * NC + c
    g_start = wid * T
    idxsA = (iA0, iA1, iA2, iA3, iA4)
    idxsB = (iB0, iB1, iB2, iB3, iB4)

    _stage_x(xc_hbm, xA, g_start, sem_x).wait()
    _compute_idx(xA, idxsA)
    _issue_gathers(tbl_hbm, idxsA, rowsA, sem_g)

    def step(i, carry):
        e = 2 * i
        ge = g_start + e
        go = ge + 1
        gn = ge + 2
        ve = ge < NCHUNKS
        vo = go < NCHUNKS
        vn = gn < NCHUNKS

        @pl.when(vo)
        def _():
            _stage_x(xc_hbm, xB, go, sem_x)

        @pl.when(ve)
        def _():
            _wait_gathers(tbl_hbm, idxsA, rowsA, sem_g)

        @pl.when(vo)
        def _():
            pltpu.make_async_copy(xc_hbm.at[pl.ds(go * FLAT, FLAT)],
                                  xB, sem_x).wait()
            _compute_idx(xB, idxsB)
            _issue_gathers(tbl_hbm, idxsB, rowsB, sem_g)

        @pl.when(ve)
        def _():
            @pl.when(i > 0)
            def _():
                pltpu.make_async_copy(
                    outA, out_hbm.at[pl.ds(0, C)], sem_oA).wait()
            _sum_chunk(rowsA, outA)
            pltpu.async_copy(outA, out_hbm.at[pl.ds(ge * C, C)], sem_oA)

        @pl.when(vn)
        def _():
            _stage_x(xc_hbm, xA, gn, sem_x)

        @pl.when(vo)
        def _():
            _wait_gathers(tbl_hbm, idxsB, rowsB, sem_g)

        @pl.when(vn)
        def _():
            pltpu.make_async_copy(xc_hbm.at[pl.ds(gn * FLAT, FLAT)],
                                  xA, sem_x).wait()
            _compute_idx(xA, idxsA)
            _issue_gathers(tbl_hbm, idxsA, rowsA, sem_g)

        @pl.when(vo)
        def _():
            @pl.when(i > 0)
            def _():
                pltpu.make_async_copy(
                    outB, out_hbm.at[pl.ds(0, C)], sem_oB).wait()
            _sum_chunk(rowsB, outB)
            pltpu.async_copy(outB, out_hbm.at[pl.ds(go * C, C)], sem_oB)
        return carry

    lax.fori_loop(0, NITER, step, 0)

    pltpu.make_async_copy(outA, out_hbm.at[pl.ds(0, C)], sem_oA).wait()
    pltpu.make_async_copy(outB, out_hbm.at[pl.ds(0, C)], sem_oB).wait()


_sc_call = pl.kernel(
    _body,
    out_type=jax.ShapeDtypeStruct((N_NODES, HIDDEN), jnp.float32),
    mesh=_mesh,
    scratch_types=(
        [pltpu.VMEM((FLAT,), jnp.int32)] * 2
        + [pltpu.VMEM((C,), jnp.int32)] * 10
        + [pltpu.VMEM((NPAIR * C, HIDDEN), jnp.float32)] * 2
        + [pltpu.VMEM((C, HIDDEN), jnp.float32)] * 2
        + [pltpu.SemaphoreType.DMA] * 4
    ),
)


def kernel(x, emb):
    pairs = [
        (emb[2 * p][:, None, :] + emb[2 * p + 1][None, :, :]).reshape(
            PAIRB, HIDDEN)
        for p in range(4)
    ]
    tbl = jnp.concatenate(pairs + [emb[8]], axis=0)
    xc = x.T.reshape(N_FEATS, NCHUNKS, C).transpose(1, 0, 2).reshape(-1)
    return _sc_call(xc, tbl)
